# baseline (speedup 1.0000x reference)
import jax
import jax.numpy as jnp
import numpy as np
from jax import lax
from jax.experimental import pallas as pl
from jax.experimental.pallas import tpu as pltpu
from jax.experimental.pallas import tpu_sc as plsc

D_MODEL = 2048
N_HEADS = 16
D_K = 128
SEQ = 2048
HI = 4
DI = 64
K_SEL = 512

_QI = HI * DI
_N_COLS = 3 * D_MODEL + _QI + DI + HI
_N_PAD = 6656
_BM = 256
_BN = 512
_HG = 8
_HGW = _HG * D_K

_theta = 1.0 / (10000.0 ** (np.arange(0, D_K, 2, dtype=np.float32) / D_K))
_freqs = np.arange(SEQ, dtype=np.float32)[:, None] * _theta[None, :]
_COS512 = np.tile(np.repeat(np.cos(_freqs).astype(np.float32), 2, axis=1),
                  (1, _BN // D_K))
_SIN512 = np.tile(np.repeat(np.sin(_freqs).astype(np.float32), 2, axis=1),
                  (1, _BN // D_K))
_R128 = np.kron(np.eye(64, dtype=np.float32),
                np.array([[0.0, 1.0], [-1.0, 0.0]], dtype=np.float32))
_R512 = np.kron(np.eye(_BN // D_K, dtype=np.float32), _R128)


def _mm(a, b):
    return jax.lax.dot_general(a, b, (((1,), (1,)), ((), ())),
                               preferred_element_type=jnp.float32)


def _p1_proj_rope(x_ref, w_ref, widx_ref, cos_ref, sin_ref, rot_ref, o_ref):
    n = pl.program_id(0)

    @pl.when(n < 8)
    def _():
        acc = _mm(x_ref[...], w_ref[...])
        rot = jnp.dot(acc, rot_ref[...], preferred_element_type=jnp.float32)
        o_ref[...] = acc * cos_ref[...] + rot * sin_ref[...]

    @pl.when((n >= 8) & (n < 12))
    def _():
        o_ref[...] = _mm(x_ref[...], w_ref[...])

    @pl.when(n == 12)
    def _():
        o_ref[...] = _mm(x_ref[...], widx_ref[...])


def _p2_scores_keys(qw_ref, ki_ref, o_ref):
    qw = qw_ref[...]
    ki = ki_ref[:, :DI]
    acc = None
    for h in range(HI):
        qh = qw[:, h * DI:(h + 1) * DI]
        d = jax.lax.dot_general(qh, ki, (((1,), (1,)), ((), ())),
                                preferred_element_type=jnp.float32)
        a = jnp.maximum(d, 0.0) * qw[:, _QI + DI + h:_QI + DI + h + 1]
        acc = a if acc is None else acc + a

    m = pl.program_id(0)
    row = 512 + m * _BM + jax.lax.broadcasted_iota(jnp.int32, (_BM, SEQ), 0)
    col = jax.lax.broadcasted_iota(jnp.int32, (_BM, SEQ), 1)
    scores = jnp.where(col > row, jnp.float32(-1e9), acc)
    z = jnp.where(scores == 0.0, jnp.float32(0.0), scores)
    bits = jax.lax.bitcast_convert_type(z, jnp.int32)
    o_ref[...] = jnp.where(bits < 0, bits ^ jnp.int32(0x7FFFFFFF), bits)


_SC_NC = 2
_SC_NW = 32
_SC_ROWS = SEQ - K_SEL
_SC_RPW = _SC_ROWS // _SC_NW
_NCH = SEQ // 16
_MININT = np.int32(-2147483648)


def _sc_select_body(keys_hbm, mask_hbm, row_v, hist_v, mask_v):
    wid = lax.axis_index("s") * _SC_NC + lax.axis_index("c")
    ones = jnp.ones((16,), jnp.int32)
    zeros = jnp.zeros((16,), jnp.int32)
    lane = lax.iota(jnp.int32, 16)

    def level(nbins, krem, bin_fn):
        nch = nbins // 16

        def zbody(i, c):
            hist_v[pl.ds(i * 16, 16)] = zeros
            return c
        lax.fori_loop(0, nch, zbody, jnp.int32(0))

        def hbody(i, c):
            u = row_v[pl.ds(i * 16, 16)] ^ _MININT
            b, msk = bin_fn(u)
            if msk is None:
                plsc.addupdate_scatter(hist_v, [b], ones)
            else:
                plsc.addupdate_scatter(hist_v, [b], ones, mask=msk)
            return c
        lax.fori_loop(0, _NCH, hbody, jnp.int32(0))

        def sbody(i, carry):
            tot, cc, above = carry
            c = nch - 1 - i
            s = jnp.sum(hist_v[pl.ds(c * 16, 16)])
            newtot = tot + s
            crossed = jnp.logical_and(newtot >= krem, cc < 0)
            return (newtot,
                    jnp.where(crossed, c, cc),
                    jnp.where(crossed, tot, above))
        _, cc, above = lax.fori_loop(
            0, nch, sbody, (jnp.int32(0), jnp.int32(-1), jnp.int32(0)))

        hh = hist_v[pl.ds(cc * 16, 16)]
        sfx = above + lax.rev(plsc.cumsum(lax.rev(hh, (0,))), (0,))
        idx = cc * 16 + lane
        cand = jnp.where(sfx >= krem, idx, jnp.int32(-1))
        b = jnp.max(cand)
        sfx_b = jnp.max(jnp.where(idx == b, sfx, jnp.int32(0)))
        h_b = jnp.max(jnp.where(idx == b, hh, jnp.int32(0)))
        return b, sfx_b - h_b

    def row_body(r, carry):
        base = (wid * _SC_RPW + r) * SEQ
        pltpu.sync_copy(keys_hbm.at[pl.ds(base, SEQ)], row_v)

        b1, a1 = level(SEQ, jnp.int32(K_SEL),
                       lambda u: (lax.shift_right_logical(u, 21), None))
        k2 = jnp.int32(K_SEL) - a1
        b2, a2 = level(
            SEQ, k2,
            lambda u: (lax.shift_right_logical(u, 10) & jnp.int32(0x7FF),
                       lax.shift_right_logical(u, 21) == b1))
        k3 = k2 - a2
        b3, a3 = level(
            1024, k3,
            lambda u: (u & jnp.int32(0x3FF),
                       jnp.logical_and(
                           lax.shift_right_logical(u, 21) == b1,
                           (lax.shift_right_logical(u, 10)
                            & jnp.int32(0x7FF)) == b2)))
        need = k3 - a3
        tau = ((b1 << 21) | (b2 << 10) | b3) ^ _MININT

        def mbody(i, cumties):
            k = row_v[pl.ds(i * 16, 16)]
            gt = k > tau
            tie = (k == tau).astype(jnp.int32)
            cs = cumties + plsc.cumsum(tie)
            sel = jnp.logical_or(gt, jnp.logical_and(tie > 0, cs <= need))
            mask_v[pl.ds(i * 16, 16)] = sel.astype(jnp.int32)
            return cumties + jnp.sum(tie)
        lax.fori_loop(0, _NCH, mbody, jnp.int32(0))

        pltpu.sync_copy(mask_v, mask_hbm.at[pl.ds(base, SEQ)])
        return carry
    lax.fori_loop(0, _SC_RPW, row_body, jnp.int32(0))


def _sc_select(keys_flat):
    return pl.kernel(
        _sc_select_body,
        out_type=jax.ShapeDtypeStruct((_SC_ROWS * SEQ,), jnp.int32),
        mesh=plsc.VectorSubcoreMesh(core_axis_name="c", subcore_axis_name="s"),
        compiler_params=pltpu.CompilerParams(needs_layout_passes=False),
        scratch_types=[
            pltpu.VMEM((SEQ,), jnp.int32),
            pltpu.VMEM((SEQ,), jnp.int32),
            pltpu.VMEM((SEQ,), jnp.int32),
        ],
    )(keys_flat)


def _p3_sdpa(q_ref, k_ref, v_ref, mask_ref, o_ref):
    keep = mask_ref[...] != 0
    for h in range(_HG):
        sl = slice(h * D_K, (h + 1) * D_K)
        s = jax.lax.dot_general(q_ref[:, sl], k_ref[:, sl],
                                (((1,), (1,)), ((), ())),
                                preferred_element_type=jnp.float32)
        s = jnp.where(keep, s / jnp.sqrt(jnp.float32(D_K)), jnp.float32(-1e30))
        mx = jnp.max(s, axis=1, keepdims=True)
        p = jnp.exp(s - mx)
        l = jnp.sum(p, axis=1, keepdims=True)
        o_ref[:, sl] = jnp.dot(p, v_ref[:, sl],
                               preferred_element_type=jnp.float32) / l


def _p3_sdpa_dense(q_ref, k_ref, v_ref, o_ref):
    for h in range(_HG):
        sl = slice(h * D_K, (h + 1) * D_K)
        s = jax.lax.dot_general(q_ref[:, sl], k_ref[:, sl],
                                (((1,), (1,)), ((), ())),
                                preferred_element_type=jnp.float32)
        s = s / jnp.sqrt(jnp.float32(D_K))
        mx = jnp.max(s, axis=1, keepdims=True)
        p = jnp.exp(s - mx)
        l = jnp.sum(p, axis=1, keepdims=True)
        o_ref[:, sl] = jnp.dot(p, v_ref[:, sl],
                               preferred_element_type=jnp.float32) / l


def _p4_matmul(a1_ref, a2_ref, a3_ref, w_ref, o_ref):
    w = w_ref[...]
    o_ref[:512, :] = _mm(a1_ref[...], w)
    o_ref[512:1024, :] = _mm(a2_ref[...], w)
    o_ref[1024:, :] = _mm(a3_ref[...], w)


def kernel(x, Wqkv, Wo, Wq_idx, Wk_idx, Ww_idx):
    b, s, _ = x.shape
    x2 = x[0]

    widx = jnp.pad(jnp.concatenate([Wq_idx, Wk_idx, Ww_idx], axis=0),
                   ((0, _BN - (_QI + DI + HI)), (0, 0)))

    proj = pl.pallas_call(
        _p1_proj_rope,
        grid=(_N_PAD // _BN,),
        in_specs=[
            pl.BlockSpec((SEQ, D_MODEL), lambda n: (0, 0)),
            pl.BlockSpec((_BN, D_MODEL), lambda n: (jnp.minimum(n, 11), 0)),
            pl.BlockSpec((_BN, D_MODEL), lambda n: (0, 0)),
            pl.BlockSpec((SEQ, _BN), lambda n: (0, 0)),
            pl.BlockSpec((SEQ, _BN), lambda n: (0, 0)),
            pl.BlockSpec((_BN, _BN), lambda n: (0, 0)),
        ],
        out_specs=pl.BlockSpec((SEQ, _BN), lambda n: (0, n)),
        out_shape=jax.ShapeDtypeStruct((SEQ, _N_PAD), jnp.float32),
    )(x2, Wqkv, widx, jnp.asarray(_COS512), jnp.asarray(_SIN512),
      jnp.asarray(_R512))

    keys = pl.pallas_call(
        _p2_scores_keys,
        grid=(_SC_ROWS // _BM,),
        in_specs=[
            pl.BlockSpec((_BM, _BN), lambda m: (2 + m, 12)),
            pl.BlockSpec((SEQ, D_K), lambda m: (0, 50)),
        ],
        out_specs=pl.BlockSpec((_BM, SEQ), lambda m: (m, 0)),
        out_shape=jax.ShapeDtypeStruct((_SC_ROWS, SEQ), jnp.int32),
    )(proj, proj)

    sel = _sc_select(keys.reshape(-1)).reshape(_SC_ROWS, SEQ)

    attn_a = pl.pallas_call(
        _p3_sdpa_dense,
        grid=(N_HEADS // _HG, 2),
        in_specs=[
            pl.BlockSpec((_BM, _HGW), lambda g, m: (m, g)),
            pl.BlockSpec((512, _HGW), lambda g, m: (0, 2 + g)),
            pl.BlockSpec((512, _HGW), lambda g, m: (0, 4 + g)),
        ],
        out_specs=pl.BlockSpec((_BM, _HGW), lambda g, m: (m, g)),
        out_shape=jax.ShapeDtypeStruct((512, D_MODEL), jnp.float32),
    )(proj, proj, proj)

    attn_b = pl.pallas_call(
        _p3_sdpa,
        grid=(N_HEADS // _HG, 2),
        in_specs=[
            pl.BlockSpec((_BM, _HGW), lambda g, m: (2 + m, g)),
            pl.BlockSpec((1024, _HGW), lambda g, m: (0, 2 + g)),
            pl.BlockSpec((1024, _HGW), lambda g, m: (0, 4 + g)),
            pl.BlockSpec((_BM, 1024), lambda g, m: (m, 0)),
        ],
        out_specs=pl.BlockSpec((_BM, _HGW), lambda g, m: (m, g)),
        out_shape=jax.ShapeDtypeStruct((512, D_MODEL), jnp.float32),
    )(proj, proj, proj, sel)

    attn_c = pl.pallas_call(
        _p3_sdpa,
        grid=(N_HEADS // _HG, 4),
        in_specs=[
            pl.BlockSpec((_BM, _HGW), lambda g, m: (4 + m, g)),
            pl.BlockSpec((SEQ, _HGW), lambda g, m: (0, 2 + g)),
            pl.BlockSpec((SEQ, _HGW), lambda g, m: (0, 4 + g)),
            pl.BlockSpec((_BM, SEQ), lambda g, m: (2 + m, 0)),
        ],
        out_specs=pl.BlockSpec((_BM, _HGW), lambda g, m: (m, g)),
        out_shape=jax.ShapeDtypeStruct((1024, D_MODEL), jnp.float32),
    )(proj, proj, proj, sel)

    out = pl.pallas_call(
        _p4_matmul,
        grid=(D_MODEL // _BN,),
        in_specs=[
            pl.BlockSpec((512, D_MODEL), lambda n: (0, 0)),
            pl.BlockSpec((512, D_MODEL), lambda n: (0, 0)),
            pl.BlockSpec((1024, D_MODEL), lambda n: (0, 0)),
            pl.BlockSpec((_BN, D_MODEL), lambda n: (n, 0)),
        ],
        out_specs=pl.BlockSpec((SEQ, _BN), lambda n: (0, n)),
        out_shape=jax.ShapeDtypeStruct((SEQ, D_MODEL), jnp.float32),
    )(attn_a, attn_b, attn_c, Wo)

    return out.reshape(b, s, D_MODEL)


# speedup vs baseline: 1.5508x; 1.5508x over previous
import jax
import jax.numpy as jnp
import numpy as np
from jax import lax
from jax.experimental import pallas as pl
from jax.experimental.pallas import tpu as pltpu
from jax.experimental.pallas import tpu_sc as plsc

D_MODEL = 2048
N_HEADS = 16
D_K = 128
SEQ = 2048
HI = 4
DI = 64
K_SEL = 512

_QI = HI * DI
_N_COLS = 3 * D_MODEL + _QI + DI + HI
_N_PAD = 6656
_BM = 256
_BN = 512
_HG = 8
_HGW = _HG * D_K

_theta = 1.0 / (10000.0 ** (np.arange(0, D_K, 2, dtype=np.float32) / D_K))
_freqs = np.arange(SEQ, dtype=np.float32)[:, None] * _theta[None, :]
_COS512 = np.tile(np.repeat(np.cos(_freqs).astype(np.float32), 2, axis=1),
                  (1, _BN // D_K))
_SIN512 = np.tile(np.repeat(np.sin(_freqs).astype(np.float32), 2, axis=1),
                  (1, _BN // D_K))
_R128 = np.kron(np.eye(64, dtype=np.float32),
                np.array([[0.0, 1.0], [-1.0, 0.0]], dtype=np.float32))
_R512 = np.kron(np.eye(_BN // D_K, dtype=np.float32), _R128)


def _mm(a, b):
    return jax.lax.dot_general(a, b, (((1,), (1,)), ((), ())),
                               preferred_element_type=jnp.float32)


def _p1_proj_rope(x_ref, w_ref, widx_ref, cos_ref, sin_ref, rot_ref, o_ref):
    n = pl.program_id(0)

    @pl.when(n < 8)
    def _():
        acc = _mm(x_ref[...], w_ref[...])
        rot = jnp.dot(acc, rot_ref[...], preferred_element_type=jnp.float32)
        o_ref[...] = acc * cos_ref[...] + rot * sin_ref[...]

    @pl.when((n >= 8) & (n < 12))
    def _():
        o_ref[...] = _mm(x_ref[...], w_ref[...])

    @pl.when(n == 12)
    def _():
        o_ref[...] = _mm(x_ref[...], widx_ref[...])


def _p2_scores_keys(qw_ref, ki_ref, o_ref):
    qw = qw_ref[...]
    ki = ki_ref[:, :DI]
    acc = None
    for h in range(HI):
        qh = qw[:, h * DI:(h + 1) * DI]
        d = jax.lax.dot_general(qh, ki, (((1,), (1,)), ((), ())),
                                preferred_element_type=jnp.float32)
        a = jnp.maximum(d, 0.0) * qw[:, _QI + DI + h:_QI + DI + h + 1]
        acc = a if acc is None else acc + a

    m = pl.program_id(0)
    row = 512 + m * _BM + jax.lax.broadcasted_iota(jnp.int32, (_BM, SEQ), 0)
    col = jax.lax.broadcasted_iota(jnp.int32, (_BM, SEQ), 1)
    scores = jnp.where(col > row, jnp.float32(-1e9), acc)
    z = jnp.where(scores == 0.0, jnp.float32(0.0), scores)
    bits = jax.lax.bitcast_convert_type(z, jnp.int32)
    o_ref[...] = jnp.where(bits < 0, bits ^ jnp.int32(0x7FFFFFFF), bits)


_SC_NC = 2
_SC_NW = 32
_SC_ROWS = SEQ - K_SEL
_SC_RPW = _SC_ROWS // _SC_NW
_NCH = SEQ // 16
_MININT = np.int32(-2147483648)


def _sc_select_body(keys_hbm, mask_hbm, row_v, hist_v, mask_v):
    wid = lax.axis_index("s") * _SC_NC + lax.axis_index("c")
    ones = jnp.ones((16,), jnp.int32)
    zeros = jnp.zeros((16,), jnp.int32)
    lane = lax.iota(jnp.int32, 16)

    def level(nbins, krem, bin_fn):
        @plsc.parallel_loop(0, nbins, 16, unroll=8)
        def _(i):
            hist_v[pl.ds(i, 16)] = zeros

        @plsc.parallel_loop(0, SEQ, 16, unroll=4)
        def _(i):
            u = row_v[pl.ds(i, 16)] ^ _MININT
            b, msk = bin_fn(u)
            if msk is None:
                plsc.addupdate_scatter(hist_v, [b], ones)
            else:
                plsc.addupdate_scatter(hist_v, [b], ones, mask=msk)

        def sbody(i, carry):
            tot, cc, above = carry
            c = nbins - 16 - i
            s = jnp.sum(hist_v[pl.ds(c, 16)])
            newtot = tot + s
            crossed = jnp.logical_and(newtot >= krem, cc < 0)
            return (newtot,
                    jnp.where(crossed, c, cc),
                    jnp.where(crossed, tot, above))
        _, cc, above = plsc.parallel_loop(
            0, nbins, 16, unroll=4,
            carry=(jnp.int32(0), jnp.int32(-1), jnp.int32(0)))(sbody)

        hh = hist_v[pl.ds(cc, 16)]
        sfx = above + lax.rev(plsc.cumsum(lax.rev(hh, (0,))), (0,))
        idx = cc + lane
        cand = jnp.where(sfx >= krem, idx, jnp.int32(-1))
        b = jnp.max(cand)
        sfx_b = jnp.max(jnp.where(idx == b, sfx, jnp.int32(0)))
        h_b = jnp.max(jnp.where(idx == b, hh, jnp.int32(0)))
        return b, sfx_b - h_b, h_b

    def row_body(r, carry):
        base = (wid * _SC_RPW + r) * SEQ
        pltpu.sync_copy(keys_hbm.at[pl.ds(base, SEQ)], row_v)

        b1, a1, _ = level(SEQ, jnp.int32(K_SEL),
                          lambda u: (lax.shift_right_logical(u, 21), None))
        k2 = jnp.int32(K_SEL) - a1
        b2, a2, _ = level(
            SEQ, k2,
            lambda u: (lax.shift_right_logical(u, 10) & jnp.int32(0x7FF),
                       lax.shift_right_logical(u, 21) == b1))
        k3 = k2 - a2
        b3, a3, nties = level(
            1024, k3,
            lambda u: (u & jnp.int32(0x3FF),
                       jnp.logical_and(
                           lax.shift_right_logical(u, 21) == b1,
                           (lax.shift_right_logical(u, 10)
                            & jnp.int32(0x7FF)) == b2)))
        need = k3 - a3
        tau = ((b1 << 21) | (b2 << 10) | b3) ^ _MININT

        @pl.when(need == nties)
        def _():
            @plsc.parallel_loop(0, SEQ, 16, unroll=4)
            def _(i):
                sel = row_v[pl.ds(i, 16)] >= tau
                mask_v[pl.ds(i, 16)] = sel.astype(jnp.int32)

        @pl.when(need != nties)
        def _():
            def mbody(i, cumties):
                k = row_v[pl.ds(i, 16)]
                tie = (k == tau).astype(jnp.int32)
                cs = cumties + plsc.cumsum(tie)
                sel = jnp.logical_or(
                    k > tau, jnp.logical_and(tie > 0, cs <= need))
                mask_v[pl.ds(i, 16)] = sel.astype(jnp.int32)
                return cumties + jnp.sum(tie)
            plsc.parallel_loop(0, SEQ, 16, unroll=2,
                               carry=jnp.int32(0))(mbody)

        pltpu.sync_copy(mask_v, mask_hbm.at[pl.ds(base, SEQ)])
        return carry
    lax.fori_loop(0, _SC_RPW, row_body, jnp.int32(0))


def _sc_select(keys_flat):
    return pl.kernel(
        _sc_select_body,
        out_type=jax.ShapeDtypeStruct((_SC_ROWS * SEQ,), jnp.int32),
        mesh=plsc.VectorSubcoreMesh(core_axis_name="c", subcore_axis_name="s"),
        compiler_params=pltpu.CompilerParams(needs_layout_passes=False),
        scratch_types=[
            pltpu.VMEM((SEQ,), jnp.int32),
            pltpu.VMEM((SEQ,), jnp.int32),
            pltpu.VMEM((SEQ,), jnp.int32),
        ],
    )(keys_flat)


def _p3_sdpa(q_ref, k_ref, v_ref, mask_ref, o_ref):
    keep = mask_ref[...] != 0
    for h in range(_HG):
        sl = slice(h * D_K, (h + 1) * D_K)
        s = jax.lax.dot_general(q_ref[:, sl], k_ref[:, sl],
                                (((1,), (1,)), ((), ())),
                                preferred_element_type=jnp.float32)
        s = jnp.where(keep, s / jnp.sqrt(jnp.float32(D_K)), jnp.float32(-1e30))
        mx = jnp.max(s, axis=1, keepdims=True)
        p = jnp.exp(s - mx)
        l = jnp.sum(p, axis=1, keepdims=True)
        o_ref[:, sl] = jnp.dot(p, v_ref[:, sl],
                               preferred_element_type=jnp.float32) / l


def _p3_sdpa_dense(q_ref, k_ref, v_ref, o_ref):
    for h in range(_HG):
        sl = slice(h * D_K, (h + 1) * D_K)
        s = jax.lax.dot_general(q_ref[:, sl], k_ref[:, sl],
                                (((1,), (1,)), ((), ())),
                                preferred_element_type=jnp.float32)
        s = s / jnp.sqrt(jnp.float32(D_K))
        mx = jnp.max(s, axis=1, keepdims=True)
        p = jnp.exp(s - mx)
        l = jnp.sum(p, axis=1, keepdims=True)
        o_ref[:, sl] = jnp.dot(p, v_ref[:, sl],
                               preferred_element_type=jnp.float32) / l


def _p4_matmul(a1_ref, a2_ref, a3_ref, w_ref, o_ref):
    w = w_ref[...]
    o_ref[:512, :] = _mm(a1_ref[...], w)
    o_ref[512:1024, :] = _mm(a2_ref[...], w)
    o_ref[1024:, :] = _mm(a3_ref[...], w)


def kernel(x, Wqkv, Wo, Wq_idx, Wk_idx, Ww_idx):
    b, s, _ = x.shape
    x2 = x[0]

    widx = jnp.pad(jnp.concatenate([Wq_idx, Wk_idx, Ww_idx], axis=0),
                   ((0, _BN - (_QI + DI + HI)), (0, 0)))

    proj = pl.pallas_call(
        _p1_proj_rope,
        grid=(_N_PAD // _BN,),
        in_specs=[
            pl.BlockSpec((SEQ, D_MODEL), lambda n: (0, 0)),
            pl.BlockSpec((_BN, D_MODEL), lambda n: (jnp.minimum(n, 11), 0)),
            pl.BlockSpec((_BN, D_MODEL), lambda n: (0, 0)),
            pl.BlockSpec((SEQ, _BN), lambda n: (0, 0)),
            pl.BlockSpec((SEQ, _BN), lambda n: (0, 0)),
            pl.BlockSpec((_BN, _BN), lambda n: (0, 0)),
        ],
        out_specs=pl.BlockSpec((SEQ, _BN), lambda n: (0, n)),
        out_shape=jax.ShapeDtypeStruct((SEQ, _N_PAD), jnp.float32),
    )(x2, Wqkv, widx, jnp.asarray(_COS512), jnp.asarray(_SIN512),
      jnp.asarray(_R512))

    keys = pl.pallas_call(
        _p2_scores_keys,
        grid=(_SC_ROWS // _BM,),
        in_specs=[
            pl.BlockSpec((_BM, _BN), lambda m: (2 + m, 12)),
            pl.BlockSpec((SEQ, D_K), lambda m: (0, 50)),
        ],
        out_specs=pl.BlockSpec((_BM, SEQ), lambda m: (m, 0)),
        out_shape=jax.ShapeDtypeStruct((_SC_ROWS, SEQ), jnp.int32),
    )(proj, proj)

    sel = _sc_select(keys.reshape(-1)).reshape(_SC_ROWS, SEQ)

    attn_a = pl.pallas_call(
        _p3_sdpa_dense,
        grid=(N_HEADS // _HG, 2),
        in_specs=[
            pl.BlockSpec((_BM, _HGW), lambda g, m: (m, g)),
            pl.BlockSpec((512, _HGW), lambda g, m: (0, 2 + g)),
            pl.BlockSpec((512, _HGW), lambda g, m: (0, 4 + g)),
        ],
        out_specs=pl.BlockSpec((_BM, _HGW), lambda g, m: (m, g)),
        out_shape=jax.ShapeDtypeStruct((512, D_MODEL), jnp.float32),
    )(proj, proj, proj)

    attn_b = pl.pallas_call(
        _p3_sdpa,
        grid=(N_HEADS // _HG, 2),
        in_specs=[
            pl.BlockSpec((_BM, _HGW), lambda g, m: (2 + m, g)),
            pl.BlockSpec((1024, _HGW), lambda g, m: (0, 2 + g)),
            pl.BlockSpec((1024, _HGW), lambda g, m: (0, 4 + g)),
            pl.BlockSpec((_BM, 1024), lambda g, m: (m, 0)),
        ],
        out_specs=pl.BlockSpec((_BM, _HGW), lambda g, m: (m, g)),
        out_shape=jax.ShapeDtypeStruct((512, D_MODEL), jnp.float32),
    )(proj, proj, proj, sel)

    attn_c = pl.pallas_call(
        _p3_sdpa,
        grid=(N_HEADS // _HG, 4),
        in_specs=[
            pl.BlockSpec((_BM, _HGW), lambda g, m: (4 + m, g)),
            pl.BlockSpec((SEQ, _HGW), lambda g, m: (0, 2 + g)),
            pl.BlockSpec((SEQ, _HGW), lambda g, m: (0, 4 + g)),
            pl.BlockSpec((_BM, SEQ), lambda g, m: (2 + m, 0)),
        ],
        out_specs=pl.BlockSpec((_BM, _HGW), lambda g, m: (m, g)),
        out_shape=jax.ShapeDtypeStruct((1024, D_MODEL), jnp.float32),
    )(proj, proj, proj, sel)

    out = pl.pallas_call(
        _p4_matmul,
        grid=(D_MODEL // _BN,),
        in_specs=[
            pl.BlockSpec((512, D_MODEL), lambda n: (0, 0)),
            pl.BlockSpec((512, D_MODEL), lambda n: (0, 0)),
            pl.BlockSpec((1024, D_MODEL), lambda n: (0, 0)),
            pl.BlockSpec((_BN, D_MODEL), lambda n: (n, 0)),
        ],
        out_specs=pl.BlockSpec((SEQ, _BN), lambda n: (0, n)),
        out_shape=jax.ShapeDtypeStruct((SEQ, D_MODEL), jnp.float32),
    )(attn_a, attn_b, attn_c, Wo)

    return out.reshape(b, s, D_MODEL)
